# Initial kernel scaffold; baseline (speedup 1.0000x reference)
import jax
import jax.numpy as jnp
from jax.experimental import pallas as pl


def kernel(pts, ray_o, azimuth, elevation):
    raise NotImplementedError("write your pallas kernel here")


# speedup vs baseline: 88.2984x; 88.2984x over previous
import functools
import math

import jax
import jax.numpy as jnp
import numpy as np
from jax import lax
from jax.experimental import pallas as pl
from jax.experimental.pallas import tpu as pltpu
from jax.experimental.pallas import tpu_sc as plsc

N_PTS = 10000
PADP = 10048
NCH = PADP // 16
NGRP = NCH // 4
R = 4096
K = 16
NTEC = 32
RPT = R // NTEC
FBASE = 1048576.0
NC = 32
BIG = 3.0e38
C2 = float(np.float32(0.866) * np.float32(0.866))
C2L = C2 * (1.0 - 1e-5)
TWO_PI = 2.0 * math.pi


def _raydir_body(az_ref, el_ref, rdx_ref, rdy_ref, rdz_ref):
    a = az_ref[...] * TWO_PI
    e = el_ref[...] * math.pi
    se = jnp.sin(e)
    rdx_ref[...] = -(jnp.cos(a) * se)
    rdy_ref[...] = -(jnp.sin(a) * se)
    rdz_ref[...] = -jnp.cos(e)


def _sc_body(px_h, py_h, pz_h, rdx_h, rdy_h, rdz_h,
             rox_h, roy_h, roz_h,
             oidx_h, ogx_h, ogy_h, ogz_h,
             pxv, pyv, pzv, dxv, dyv, dzv, n2v,
             rdxv, rdyv, rdzv, roxv, royv, rozv,
             sti, stx, sty, stz):
    cid = lax.axis_index("c")
    sid = lax.axis_index("s")
    wid = sid * 2 + cid
    base = wid * RPT
    pltpu.sync_copy(px_h, pxv)
    pltpu.sync_copy(py_h, pyv)
    pltpu.sync_copy(pz_h, pzv)
    for src, dst in ((rdx_h, rdxv), (rdy_h, rdyv), (rdz_h, rdzv),
                     (rox_h, roxv), (roy_h, royv), (roz_h, rozv)):
        pltpu.sync_copy(src.at[pl.ds(base, RPT)], dst)

    iota = lax.iota(jnp.int32, 16)

    def ftr_body(f, _):
        jb = f * 8
        jv0 = jnp.full((16,), jb, jnp.int32)
        ox = plsc.load_gather(roxv, [jv0])
        oy = plsc.load_gather(royv, [jv0])
        oz = plsc.load_gather(rozv, [jv0])

        def diff_body(g, _):
            for q in range(4):
                b = g * 64 + q * 16
                dx = pxv[pl.ds(b, 16)] - ox
                dy = pyv[pl.ds(b, 16)] - oy
                dz = pzv[pl.ds(b, 16)] - oz
                dxv[pl.ds(b, 16)] = dx
                dyv[pl.ds(b, 16)] = dy
                dzv[pl.ds(b, 16)] = dz
                n2v[pl.ds(b, 16)] = dx * dx + dy * dy + dz * dz
            return 0

        lax.fori_loop(0, NGRP, diff_body, 0)
        for q in range(3):
            n2v[pl.ds(N_PTS + 16 * q, 16)] = jnp.full((16,), 1e30,
                                                      jnp.float32)

        def ray_body(r, _):
            jv = jnp.full((16,), jb + r, jnp.int32)
            rdx = plsc.load_gather(rdxv, [jv])
            rdy = plsc.load_gather(rdyv, [jv])
            rdz = plsc.load_gather(rdzv, [jv])
            return _ray_topk(r, rdx, rdy, rdz,
                             pxv, pyv, pzv, dxv, dyv, dzv, n2v,
                             iota, sti, stx, sty, stz)

        lax.fori_loop(0, 8, ray_body, 0)
        for st, oh in ((sti, oidx_h), (stx, ogx_h), (sty, ogy_h),
                       (stz, ogz_h)):
            pltpu.sync_copy(st, oh.at[pl.ds(base + jb, 8)])
        return 0

    def _ray_topk(j, rdx, rdy, rdz, pxv, pyv, pzv,
                  dxv, dyv, dzv, n2v, iota, sti, stx, sty, stz):

        def chunk_key(b):
            dx = dxv[pl.ds(b, 16)]
            dy = dyv[pl.ds(b, 16)]
            dz = dzv[pl.ds(b, 16)]
            n2 = n2v[pl.ds(b, 16)]
            dt = rdx * dx + rdy * dy + rdz * dz
            dt2 = dt * dt
            projsq = n2 - dt2
            pidx = iota + b
            cone = (dt >= 0.0) & (dt2 >= C2L * n2)
            key = jnp.where(cone, projsq, FBASE + pidx.astype(jnp.float32))
            return key, pidx

        def pairsplit(ak, ai, bk, bi):
            rk = lax.rev(bk, (0,))
            ri = lax.rev(bi, (0,))
            take = ak < rk
            lo = (jnp.where(take, ak, rk), jnp.where(take, ai, ri))
            hi = (jnp.where(take, rk, ak), jnp.where(take, ri, ai))
            return lo, hi

        def group_body(g, carry):
            lk, li, hk, hi = carry
            b = g * 64
            s = []
            for q in range(4):
                kq, iq = chunk_key(b + 16 * q)
                s.append(plsc.sort_key_val(kq, iq))
            m01, _ = pairsplit(s[0][0], s[0][1], s[1][0], s[1][1])
            m23, _ = pairsplit(s[2][0], s[2][1], s[3][0], s[3][1])
            t01 = plsc.sort_key_val(*m01)
            t23 = plsc.sort_key_val(*m23)
            m, _ = pairsplit(t01[0], t01[1], t23[0], t23[1])
            tm = plsc.sort_key_val(*m)
            mlo, spill = pairsplit(tm[0], tm[1], lk, li)
            nlk, nli = plsc.sort_key_val(*mlo)
            sp = plsc.sort_key_val(*spill)
            mhi, _ = pairsplit(sp[0], sp[1], hk, hi)
            nhk, nhi = plsc.sort_key_val(*mhi)
            return (nlk, nli, nhk, nhi)

        lk, li, hk, hi = lax.fori_loop(
            0, NGRP, group_body,
            (jnp.full((16,), BIG, jnp.float32),
             jnp.zeros((16,), jnp.int32),
             jnp.full((16,), BIG, jnp.float32),
             jnp.zeros((16,), jnp.int32)))
        for col, i16 in ((0, li), (1, hi)):
            sti[j, pl.ds(col * 16, 16)] = i16
            stx[j, pl.ds(col * 16, 16)] = plsc.load_gather(pxv, [i16])
            sty[j, pl.ds(col * 16, 16)] = plsc.load_gather(pyv, [i16])
            stz[j, pl.ds(col * 16, 16)] = plsc.load_gather(pzv, [i16])
        return 0

    lax.fori_loop(0, RPT // 8, ftr_body, 0)


def _acos(x):
    ax = jnp.abs(x)
    p = -0.0012624911
    for c in (0.0066700901, -0.0170881256, 0.0308918810, -0.0501743046,
              0.0889789874, -0.2145988016, 1.5707963050):
        p = p * ax + c
    r = jnp.sqrt(jnp.maximum(1.0 - ax, 0.0)) * p
    return jnp.where(x < 0.0, math.pi - r, r)


def _epilogue_body(gx_ref, gy_ref, gz_ref, idx_ref,
                   ox_ref, oy_ref, oz_ref, rdx_ref, rdy_ref, rdz_ref,
                   vx_ref, vy_ref, vz_ref, dist_ref, az_ref, el_ref,
                   npd_ref, oidx_ref):
    dx = gx_ref[...] - ox_ref[...]
    dy = gy_ref[...] - oy_ref[...]
    dz = gz_ref[...] - oz_ref[...]
    nd = jnp.sqrt(dx * dx + dy * dy + dz * dz)
    den = jnp.maximum(nd, 1e-12)
    rdx = rdx_ref[...]
    rdy = rdy_ref[...]
    rdz = rdz_ref[...]
    cos = rdx * (dx / den) + rdy * (dy / den) + rdz * (dz / den)
    proj = jnp.sqrt(jnp.clip(1.0 - cos * cos, 1e-12, None)) * nd
    v = jnp.where(cos < 0.866, 1e8, proj)
    idxf = idx_ref[...].astype(jnp.float32)
    lastv = jnp.full((1, R), -1.0, jnp.float32)
    lasti = jnp.full((1, R), -1.0, jnp.float32)
    for r in range(K):
        elig = (v > lastv) | ((v == lastv) & (idxf > lasti))
        c = jnp.where(elig, v, BIG)
        minv = jnp.min(c, axis=0, keepdims=True)
        ci = jnp.where(c == minv, idxf, BIG)
        mini = jnp.min(ci, axis=0, keepdims=True)
        sel = (c == minv) & (idxf == mini)
        selx = jnp.sum(jnp.where(sel, gx_ref[...], 0.0), axis=0)
        sely = jnp.sum(jnp.where(sel, gy_ref[...], 0.0), axis=0)
        selz = jnp.sum(jnp.where(sel, gz_ref[...], 0.0), axis=0)
        npd_ref[r, :] = minv[0]
        oidx_ref[r, :] = mini[0].astype(jnp.int32)
        vx_ref[r, :] = selx
        vy_ref[r, :] = sely
        vz_ref[r, :] = selz
        lastv, lasti = minv, mini
    wx = vx_ref[...] - ox_ref[...]
    wy = vy_ref[...] - oy_ref[...]
    wz = vz_ref[...] - oz_ref[...]
    nd16 = jnp.sqrt(wx * wx + wy * wy + wz * wz)
    rx = rdx * nd16
    ry = rdy * nd16
    rz = rdz * nd16
    dist = jnp.sqrt(rx * rx + ry * ry + rz * rz)
    dd = jnp.maximum(dist, 1e-12)
    vx = rx / dd
    vy = ry / dd
    vz = rz / dd
    elev = _acos(jnp.clip(vz, -1.0, 1.0))
    sin_el = jnp.sin(elev)
    sin_el_inv = jnp.where(jnp.abs(sin_el) < 1e-5, 0.0, 1.0 / sin_el)
    azm = _acos(jnp.clip(vx * sin_el_inv, -1.0, 1.0))
    azm = jnp.where(vy < 0.0, TWO_PI - azm, azm)
    vx_ref[...] = vx
    vy_ref[...] = vy
    vz_ref[...] = vz
    dist_ref[...] = dist
    az_ref[...] = azm
    el_ref[...] = elev


def kernel(pts, ray_o, azimuth, elevation):
    f32 = jnp.float32
    px = jnp.pad(pts[:, 0], (0, PADP - N_PTS))
    py = jnp.pad(pts[:, 1], (0, PADP - N_PTS))
    pz = jnp.pad(pts[:, 2], (0, PADP - N_PTS))
    az2 = azimuth.reshape(32, 128)
    el2 = elevation.reshape(32, 128)

    rdx, rdy, rdz = pl.pallas_call(
        _raydir_body,
        out_shape=[jax.ShapeDtypeStruct((32, 128), f32)] * 3,
    )(az2, el2)

    rd = jnp.stack([rdx.reshape(R), rdy.reshape(R), rdz.reshape(R)], axis=-1)
    ro = jnp.broadcast_to(ray_o, (512, 8, 3)).reshape(R, 3)

    mesh = plsc.VectorSubcoreMesh(core_axis_name="c", subcore_axis_name="s")
    sc = pl.kernel(
        _sc_body,
        out_type=[
            jax.ShapeDtypeStruct((R, NC), jnp.int32),
            jax.ShapeDtypeStruct((R, NC), f32),
            jax.ShapeDtypeStruct((R, NC), f32),
            jax.ShapeDtypeStruct((R, NC), f32),
        ],
        mesh=mesh,
        scratch_types=[
            pltpu.VMEM((PADP,), f32),
            pltpu.VMEM((PADP,), f32),
            pltpu.VMEM((PADP,), f32),
            pltpu.VMEM((PADP,), f32),
            pltpu.VMEM((PADP,), f32),
            pltpu.VMEM((PADP,), f32),
            pltpu.VMEM((PADP,), f32),
            pltpu.VMEM((RPT,), f32),
            pltpu.VMEM((RPT,), f32),
            pltpu.VMEM((RPT,), f32),
            pltpu.VMEM((RPT,), f32),
            pltpu.VMEM((RPT,), f32),
            pltpu.VMEM((RPT,), f32),
            pltpu.VMEM((8, NC), jnp.int32),
            pltpu.VMEM((8, NC), f32),
            pltpu.VMEM((8, NC), f32),
            pltpu.VMEM((8, NC), f32),
        ],
        compiler_params=pltpu.CompilerParams(needs_layout_passes=False),
    )
    oidx, ogx, ogy, ogz = sc(
        px, py, pz,
        rd[:, 0], rd[:, 1], rd[:, 2],
        ro[:, 0], ro[:, 1], ro[:, 2])

    row = lambda a: a.reshape(1, R)
    outs = pl.pallas_call(
        _epilogue_body,
        out_shape=[jax.ShapeDtypeStruct((K, R), f32)] * 7
        + [jax.ShapeDtypeStruct((K, R), jnp.int32)],
    )(ogx.T, ogy.T, ogz.T, oidx.T,
      row(ro[:, 0]), row(ro[:, 1]), row(ro[:, 2]),
      row(rd[:, 0]), row(rd[:, 1]), row(rd[:, 2]))
    vx, vy, vz, dist, azm, elev, npd, idx16 = outs

    def back(a):
        return a.T.reshape(512, 8, K)

    info = jnp.stack(
        [back(vx), back(vy), back(vz), back(dist), back(azm), back(elev)],
        axis=-1)
    npd_out = back(npd)
    idx_out = idx16.T.reshape(512, 8, K)
    hit_sky = npd_out >= (1e8 - 1)
    return (info, npd_out, idx_out, hit_sky)


# speedup vs baseline: 93.0815x; 1.0542x over previous
import math

import jax
import jax.numpy as jnp
import numpy as np
from jax import lax
from jax.experimental import pallas as pl
from jax.experimental.pallas import tpu as pltpu
from jax.experimental.pallas import tpu_sc as plsc

N_PTS = 10000
PADP = 10048
NCH = PADP // 16
NGRP = NCH // 4
R = 4096
K = 16
NTEC = 32
RPT = R // NTEC
FBASE = 1048576.0
NC = 32
BIG = 3.0e38
C2 = float(np.float32(0.866) * np.float32(0.866))
C2L = C2 * (1.0 - 1e-5)
TWO_PI = 2.0 * math.pi


def _raydir_body(az_ref, el_ref, rdx_ref, rdy_ref, rdz_ref):
    a = az_ref[...] * TWO_PI
    e = el_ref[...] * math.pi
    se = jnp.sin(e)
    rdx_ref[...] = -(jnp.cos(a) * se)
    rdy_ref[...] = -(jnp.sin(a) * se)
    rdz_ref[...] = -jnp.cos(e)


def _sc_body(px_h, py_h, pz_h, rdx_h, rdy_h, rdz_h,
             rox_h, roy_h, roz_h,
             oidx_h, ogx_h, ogy_h, ogz_h,
             pxv, pyv, pzv, dxv, dyv, dzv, n2v,
             rdxv, rdyv, rdzv, roxv, royv, rozv,
             sti, stx, sty, stz):
    cid = lax.axis_index("c")
    sid = lax.axis_index("s")
    wid = sid * 2 + cid
    base = wid * RPT
    pltpu.sync_copy(px_h, pxv)
    pltpu.sync_copy(py_h, pyv)
    pltpu.sync_copy(pz_h, pzv)
    for src, dst in ((rdx_h, rdxv), (rdy_h, rdyv), (rdz_h, rdzv),
                     (rox_h, roxv), (roy_h, royv), (roz_h, rozv)):
        pltpu.sync_copy(src.at[pl.ds(base, RPT)], dst)

    iota = lax.iota(jnp.int32, 16)

    def ftr_body(f, _):
        jb = f * 8
        jv0 = jnp.full((16,), jb, jnp.int32)
        ox = plsc.load_gather(roxv, [jv0])
        oy = plsc.load_gather(royv, [jv0])
        oz = plsc.load_gather(rozv, [jv0])

        def diff_body(g, _):
            for q in range(4):
                b = g * 64 + q * 16
                dx = pxv[pl.ds(b, 16)] - ox
                dy = pyv[pl.ds(b, 16)] - oy
                dz = pzv[pl.ds(b, 16)] - oz
                dxv[pl.ds(b, 16)] = dx
                dyv[pl.ds(b, 16)] = dy
                dzv[pl.ds(b, 16)] = dz
                n2v[pl.ds(b, 16)] = dx * dx + dy * dy + dz * dz
            return 0

        lax.fori_loop(0, NGRP, diff_body, 0)
        for q in range(3):
            n2v[pl.ds(N_PTS + 16 * q, 16)] = jnp.full((16,), 1e30,
                                                      jnp.float32)

        def ray_body(r, _):
            jv = jnp.full((16,), jb + r, jnp.int32)
            rdx = plsc.load_gather(rdxv, [jv])
            rdy = plsc.load_gather(rdyv, [jv])
            rdz = plsc.load_gather(rdzv, [jv])
            return _ray_topk(r, rdx, rdy, rdz,
                             pxv, pyv, pzv, dxv, dyv, dzv, n2v,
                             iota, sti, stx, sty, stz)

        lax.fori_loop(0, 8, ray_body, 0)
        for st, oh in ((sti, oidx_h), (stx, ogx_h), (sty, ogy_h),
                       (stz, ogz_h)):
            pltpu.sync_copy(st, oh.at[pl.ds(base + jb, 8)])
        return 0

    def _ray_topk(j, rdx, rdy, rdz, pxv, pyv, pzv,
                  dxv, dyv, dzv, n2v, iota, sti, stx, sty, stz):

        def chunk_key(b, pidx, synth):
            dx = dxv[pl.ds(b, 16)]
            dy = dyv[pl.ds(b, 16)]
            dz = dzv[pl.ds(b, 16)]
            n2 = n2v[pl.ds(b, 16)]
            dt = rdx * dx + rdy * dy + rdz * dz
            dt2 = dt * dt
            projsq = n2 - dt2
            cone = (dt >= 0.0) & (dt2 >= C2L * n2)
            key = jnp.where(cone, projsq, synth)
            return key, pidx

        def pairsplit(ak, ai, bk, bi):
            rk = lax.rev(bk, (0,))
            ri = lax.rev(bi, (0,))
            take = ak < rk
            lo = (jnp.where(take, ak, rk), jnp.where(take, ai, ri))
            hi = (jnp.where(take, rk, ak), jnp.where(take, ri, ai))
            return lo, hi

        def group_body(g, carry):
            lk, li, hk, hi, ibase, sbase = carry
            b = g * 64
            s = []
            for q in range(4):
                kq, iq = chunk_key(b + 16 * q, ibase + 16 * q,
                                   sbase + 16.0 * q)
                s.append(plsc.sort_key_val(kq, iq))
            m01, _ = pairsplit(s[0][0], s[0][1], s[1][0], s[1][1])
            m23, _ = pairsplit(s[2][0], s[2][1], s[3][0], s[3][1])
            t01 = plsc.sort_key_val(*m01)
            t23 = plsc.sort_key_val(*m23)
            m, _ = pairsplit(t01[0], t01[1], t23[0], t23[1])
            tm = plsc.sort_key_val(*m)
            mlo, spill = pairsplit(tm[0], tm[1], lk, li)
            nlk, nli = plsc.sort_key_val(*mlo)
            sp = plsc.sort_key_val(*spill)
            mhi, _ = pairsplit(sp[0], sp[1], hk, hi)
            nhk, nhi = plsc.sort_key_val(*mhi)
            return (nlk, nli, nhk, nhi, ibase + 64, sbase + 64.0)

        lk, li, hk, hi, _, _ = lax.fori_loop(
            0, NGRP, group_body,
            (jnp.full((16,), BIG, jnp.float32),
             jnp.zeros((16,), jnp.int32),
             jnp.full((16,), BIG, jnp.float32),
             jnp.zeros((16,), jnp.int32),
             iota,
             FBASE + iota.astype(jnp.float32)))
        for col, i16 in ((0, li), (1, hi)):
            sti[j, pl.ds(col * 16, 16)] = i16
            stx[j, pl.ds(col * 16, 16)] = plsc.load_gather(pxv, [i16])
            sty[j, pl.ds(col * 16, 16)] = plsc.load_gather(pyv, [i16])
            stz[j, pl.ds(col * 16, 16)] = plsc.load_gather(pzv, [i16])
        return 0

    lax.fori_loop(0, RPT // 8, ftr_body, 0)


def _acos(x):
    ax = jnp.abs(x)
    p = -0.0012624911
    for c in (0.0066700901, -0.0170881256, 0.0308918810, -0.0501743046,
              0.0889789874, -0.2145988016, 1.5707963050):
        p = p * ax + c
    r = jnp.sqrt(jnp.maximum(1.0 - ax, 0.0)) * p
    return jnp.where(x < 0.0, math.pi - r, r)


def _epilogue_body(gx_ref, gy_ref, gz_ref, idx_ref,
                   ox_ref, oy_ref, oz_ref, rdx_ref, rdy_ref, rdz_ref,
                   vx_ref, vy_ref, vz_ref, dist_ref, az_ref, el_ref,
                   npd_ref, oidx_ref):
    dx = gx_ref[...] - ox_ref[...]
    dy = gy_ref[...] - oy_ref[...]
    dz = gz_ref[...] - oz_ref[...]
    nd = jnp.sqrt(dx * dx + dy * dy + dz * dz)
    den = jnp.maximum(nd, 1e-12)
    rdx = rdx_ref[...]
    rdy = rdy_ref[...]
    rdz = rdz_ref[...]
    cos = rdx * (dx / den) + rdy * (dy / den) + rdz * (dz / den)
    proj = jnp.sqrt(jnp.clip(1.0 - cos * cos, 1e-12, None)) * nd
    v = jnp.where(cos < 0.866, 1e8, proj)
    idxf = idx_ref[...].astype(jnp.float32)
    lastv = jnp.full((1, R), -1.0, jnp.float32)
    lasti = jnp.full((1, R), -1.0, jnp.float32)
    for r in range(K):
        elig = (v > lastv) | ((v == lastv) & (idxf > lasti))
        c = jnp.where(elig, v, BIG)
        minv = jnp.min(c, axis=0, keepdims=True)
        ci = jnp.where(c == minv, idxf, BIG)
        mini = jnp.min(ci, axis=0, keepdims=True)
        sel = (c == minv) & (idxf == mini)
        selx = jnp.sum(jnp.where(sel, gx_ref[...], 0.0), axis=0)
        sely = jnp.sum(jnp.where(sel, gy_ref[...], 0.0), axis=0)
        selz = jnp.sum(jnp.where(sel, gz_ref[...], 0.0), axis=0)
        npd_ref[r, :] = minv[0]
        oidx_ref[r, :] = mini[0].astype(jnp.int32)
        vx_ref[r, :] = selx
        vy_ref[r, :] = sely
        vz_ref[r, :] = selz
        lastv, lasti = minv, mini
    wx = vx_ref[...] - ox_ref[...]
    wy = vy_ref[...] - oy_ref[...]
    wz = vz_ref[...] - oz_ref[...]
    nd16 = jnp.sqrt(wx * wx + wy * wy + wz * wz)
    rx = rdx * nd16
    ry = rdy * nd16
    rz = rdz * nd16
    dist = jnp.sqrt(rx * rx + ry * ry + rz * rz)
    dd = jnp.maximum(dist, 1e-12)
    vx = rx / dd
    vy = ry / dd
    vz = rz / dd
    elev = _acos(jnp.clip(vz, -1.0, 1.0))
    sin_el = jnp.sin(elev)
    sin_el_inv = jnp.where(jnp.abs(sin_el) < 1e-5, 0.0, 1.0 / sin_el)
    azm = _acos(jnp.clip(vx * sin_el_inv, -1.0, 1.0))
    azm = jnp.where(vy < 0.0, TWO_PI - azm, azm)
    vx_ref[...] = vx
    vy_ref[...] = vy
    vz_ref[...] = vz
    dist_ref[...] = dist
    az_ref[...] = azm
    el_ref[...] = elev


def kernel(pts, ray_o, azimuth, elevation):
    f32 = jnp.float32
    px = jnp.pad(pts[:, 0], (0, PADP - N_PTS))
    py = jnp.pad(pts[:, 1], (0, PADP - N_PTS))
    pz = jnp.pad(pts[:, 2], (0, PADP - N_PTS))
    az2 = azimuth.reshape(32, 128)
    el2 = elevation.reshape(32, 128)

    rdx, rdy, rdz = pl.pallas_call(
        _raydir_body,
        out_shape=[jax.ShapeDtypeStruct((32, 128), f32)] * 3,
    )(az2, el2)

    rd = jnp.stack([rdx.reshape(R), rdy.reshape(R), rdz.reshape(R)], axis=-1)
    ro = jnp.broadcast_to(ray_o, (512, 8, 3)).reshape(R, 3)

    mesh = plsc.VectorSubcoreMesh(core_axis_name="c", subcore_axis_name="s")
    sc = pl.kernel(
        _sc_body,
        out_type=[
            jax.ShapeDtypeStruct((R, NC), jnp.int32),
            jax.ShapeDtypeStruct((R, NC), f32),
            jax.ShapeDtypeStruct((R, NC), f32),
            jax.ShapeDtypeStruct((R, NC), f32),
        ],
        mesh=mesh,
        scratch_types=[
            pltpu.VMEM((PADP,), f32),
            pltpu.VMEM((PADP,), f32),
            pltpu.VMEM((PADP,), f32),
            pltpu.VMEM((PADP,), f32),
            pltpu.VMEM((PADP,), f32),
            pltpu.VMEM((PADP,), f32),
            pltpu.VMEM((PADP,), f32),
            pltpu.VMEM((RPT,), f32),
            pltpu.VMEM((RPT,), f32),
            pltpu.VMEM((RPT,), f32),
            pltpu.VMEM((RPT,), f32),
            pltpu.VMEM((RPT,), f32),
            pltpu.VMEM((RPT,), f32),
            pltpu.VMEM((8, NC), jnp.int32),
            pltpu.VMEM((8, NC), f32),
            pltpu.VMEM((8, NC), f32),
            pltpu.VMEM((8, NC), f32),
        ],
        compiler_params=pltpu.CompilerParams(needs_layout_passes=False),
    )
    oidx, ogx, ogy, ogz = sc(
        px, py, pz,
        rd[:, 0], rd[:, 1], rd[:, 2],
        ro[:, 0], ro[:, 1], ro[:, 2])

    row = lambda a: a.reshape(1, R)
    outs = pl.pallas_call(
        _epilogue_body,
        out_shape=[jax.ShapeDtypeStruct((K, R), f32)] * 7
        + [jax.ShapeDtypeStruct((K, R), jnp.int32)],
    )(ogx.T, ogy.T, ogz.T, oidx.T,
      row(ro[:, 0]), row(ro[:, 1]), row(ro[:, 2]),
      row(rd[:, 0]), row(rd[:, 1]), row(rd[:, 2]))
    vx, vy, vz, dist, azm, elev, npd, idx16 = outs

    def back(a):
        return a.T.reshape(512, 8, K)

    info = jnp.stack(
        [back(vx), back(vy), back(vz), back(dist), back(azm), back(elev)],
        axis=-1)
    npd_out = back(npd)
    idx_out = idx16.T.reshape(512, 8, K)
    hit_sky = npd_out >= (1e8 - 1)
    return (info, npd_out, idx_out, hit_sky)
